# baseline (speedup 1.0000x reference)
import functools

import jax
import jax.numpy as jnp
from jax import lax
from jax.experimental import pallas as pl
from jax.experimental.pallas import tpu as pltpu
from jax.experimental.pallas import tpu_sc as plsc

D_MODEL = 64
IDX_MINOR = 128


def _table_pack_kernel(t_ref, o_ref):
    a = t_ref[...]
    h = jnp.concatenate([a[:, :256], a[:, 256:]], axis=0)
    o_ref[...] = h.T


def _table_pack(tbl_t):
    d, v = tbl_t.shape
    nblk = (v + 511) // 512
    return pl.pallas_call(
        _table_pack_kernel,
        grid=(nblk,),
        in_specs=[pl.BlockSpec((d, 512), lambda b: (0, b))],
        out_specs=pl.BlockSpec((256, 128), lambda b: (b, 0)),
        out_shape=jax.ShapeDtypeStruct((nblk * 256, 128), jnp.float32),
    )(tbl_t)


def _unpack_kernel(z_ref, o_ref):
    g = z_ref[...].T
    o_ref[0] = jnp.concatenate([g[:64, :], g[64:, :]], axis=1)


def _unpack(z2, nj, ni):
    iblks = ni // 512
    return pl.pallas_call(
        _unpack_kernel,
        grid=(nj, iblks),
        in_specs=[pl.BlockSpec((256, 128), lambda j, ib: (j * iblks + ib, 0))],
        out_specs=pl.BlockSpec((1, 64, 512), lambda j, ib: (j, 0, ib)),
        out_shape=jax.ShapeDtypeStruct((nj, D_MODEL, ni), jnp.float32),
    )(z2)


def _make_gather(B: int):
    info = plsc.get_sparse_core_info()
    nw = info.num_cores * info.num_subcores
    K = 5
    NB = 2
    C = K * IDX_MINOR
    b_per_w = B // nw
    chunks_per_w = b_per_w // C
    G = chunks_per_w // NB
    assert b_per_w % C == 0 and B % nw == 0 and chunks_per_w % NB == 0

    mesh = plsc.VectorSubcoreMesh(core_axis_name="c", subcore_axis_name="s")

    @functools.partial(
        pl.kernel,
        mesh=mesh,
        out_type=jax.ShapeDtypeStruct((B, D_MODEL), jnp.float32),
        scratch_types=[
            [pltpu.VMEM((K, IDX_MINOR), jnp.int32) for _ in range(NB)],
            [pltpu.VMEM((C, D_MODEL), jnp.float32) for _ in range(NB)],
            [pltpu.SemaphoreType.DMA for _ in range(NB)],
            [pltpu.SemaphoreType.DMA for _ in range(NB)],
        ],
        compiler_params=pltpu.CompilerParams(use_tc_tiling_on_sc=False),
    )
    def gather_kernel(idx_hbm, table_hbm, out_hbm, idx_v, rows_v, gsem, wsem):
        wid = lax.axis_index("s") * info.num_cores + lax.axis_index("c")
        idx_row0 = wid * (b_per_w // IDX_MINOR)
        out_row0 = wid * b_per_w

        def fire(i, b):
            pltpu.sync_copy(idx_hbm.at[pl.ds(idx_row0 + i * K, K)], idx_v[b])
            for j in range(K):
                pltpu.async_copy(
                    table_hbm.at[idx_v[b].at[j]],
                    rows_v[b].at[pl.ds(j * IDX_MINOR, IDX_MINOR)],
                    gsem[b],
                )

        def drain_gather(b):
            for j in range(K):
                pltpu.make_async_copy(
                    table_hbm.at[idx_v[b].at[j]],
                    rows_v[b].at[pl.ds(j * IDX_MINOR, IDX_MINOR)],
                    gsem[b],
                ).wait()

        def write_out(i, b):
            return pltpu.async_copy(
                rows_v[b], out_hbm.at[pl.ds(out_row0 + i * C, C)], wsem[b]
            )

        for b in range(NB):
            fire(b, b)

        def body(g, carry):
            for b in range(NB):
                i = g * NB + b
                drain_gather(b)
                w = write_out(i, b)
                w.wait()
                fire(i + NB, b)
            return carry

        lax.fori_loop(0, G - 1, body, 0)

        for b in range(NB):
            i = (G - 1) * NB + b
            drain_gather(b)
            write_out(i, b).wait()

    return gather_kernel


def _half_interleave_pos(t):
    return jnp.where(t < 256, 2 * t, 2 * t - 511)


def kernel(x, emb_weight):
    ni, nj = x.shape
    v, d = emb_weight.shape
    B = x.size
    xt = x.T.reshape(B).astype(jnp.int32)
    tbl_t = emb_weight.T

    t = xt & 511
    ix = (xt - t) + _half_interleave_pos(t)
    ix = ix.reshape(B // 512, 2, 256).transpose(0, 2, 1).reshape(B // IDX_MINOR, IDX_MINOR)

    t2 = _table_pack(tbl_t)
    tbl_lin = t2.reshape(-1, d)
    out_lin = _make_gather(B)(ix, tbl_lin)
    z2 = out_lin.reshape(B // 2, 128)
    y = _unpack(z2, nj, ni)
    return jnp.transpose(y, (2, 0, 1))
